# baseline (speedup 1.0000x reference)
import jax
import jax.numpy as jnp
from jax.experimental import pallas as pl
from jax.experimental.pallas import tpu as pltpu

_K = 10


def _retrieve_body(qk_ref, mk_ref, mpos_ref, mask_ref,
                   topi_ref, pos_ref, mask_out_ref):
    bq = qk_ref.shape[0]
    m = mk_ref.shape[0]
    s = mask_ref.shape[1]

    qk = qk_ref[...]
    mk = mk_ref[...]
    qn = qk / (jnp.sqrt(jnp.sum(qk * qk, axis=1, keepdims=True)) + 1e-8)
    mn = mk / (jnp.sqrt(jnp.sum(mk * mk, axis=1, keepdims=True)) + 1e-8)
    sims = jax.lax.dot_general(
        qn, mn, (((1,), (1,)), ((), ())), preferred_element_type=jnp.float32)

    iota = jax.lax.broadcasted_iota(jnp.int32, (bq, m), 1)
    mpos = mpos_ref[...]
    cur = sims
    idx_cols = []
    pos_cols = []
    for _ in range(_K):
        mx = jnp.max(cur, axis=1, keepdims=True)
        hit = cur == mx
        sel = jnp.min(jnp.where(hit, iota, m), axis=1, keepdims=True)
        idx_cols.append(sel)
        here = iota == sel
        pos_cols.append(jnp.sum(jnp.where(here, mpos, 0.0), axis=1, keepdims=True))
        cur = jnp.where(here, -jnp.inf, cur)

    topi_ref[...] = jnp.concatenate(idx_cols, axis=1)
    mask_out_ref[:, :_K] = jnp.ones((bq, _K), mask_out_ref.dtype)
    mask_out_ref[:, _K:] = mask_ref[...]
    pos_ref[:, :s] = jax.lax.broadcasted_iota(jnp.int32, (bq, s), 1).astype(jnp.float32)
    pos_ref[:, s:] = jnp.concatenate(pos_cols, axis=1)


def _concat_body(topi_ref, k_ref, v_ref, mk_ref, mv_ref, ok_ref, ov_ref):
    b = pl.program_id(1)
    ok_ref[0, 0, _K:, :] = k_ref[0, 0, :, :]
    ov_ref[0, 0, _K:, :] = v_ref[0, 0, :, :]
    base = b * _K
    for j in range(_K):
        idx = topi_ref[base + j]
        ok_ref[0, 0, pl.ds(j, 1), :] = mk_ref[pl.ds(idx, 1), :]
        ov_ref[0, 0, pl.ds(j, 1), :] = mv_ref[pl.ds(idx, 1), :]


def kernel(inputs, q, k, v, attention_mask, mem_keys, mem_values,
           mem_positions, seq_len_q):
    b, h, s, dh = q.shape
    m = mem_keys.shape[0]

    query_key = k[:, :, s - 1, :].reshape(b, h * dh)
    mpos2 = mem_positions.reshape(1, m)

    topi, positions_k, mask_aug = pl.pallas_call(
        _retrieve_body,
        out_shape=(
            jax.ShapeDtypeStruct((b, _K), jnp.int32),
            jax.ShapeDtypeStruct((b, s + _K), jnp.float32),
            jax.ShapeDtypeStruct((b, s + _K), attention_mask.dtype),
        ),
    )(query_key, mem_keys, mpos2, attention_mask)

    topi_flat = topi.reshape(b * _K)

    grid_spec = pltpu.PrefetchScalarGridSpec(
        num_scalar_prefetch=1,
        grid=(h, b),
        in_specs=[
            pl.BlockSpec((1, 1, s, dh), lambda hh, bb, t: (bb, hh, 0, 0)),
            pl.BlockSpec((1, 1, s, dh), lambda hh, bb, t: (bb, hh, 0, 0)),
            pl.BlockSpec((m, dh), lambda hh, bb, t: (0, hh)),
            pl.BlockSpec((m, dh), lambda hh, bb, t: (0, hh)),
        ],
        out_specs=[
            pl.BlockSpec((1, 1, _K + s, dh), lambda hh, bb, t: (bb, hh, 0, 0)),
            pl.BlockSpec((1, 1, _K + s, dh), lambda hh, bb, t: (bb, hh, 0, 0)),
        ],
    )
    k_aug, v_aug = pl.pallas_call(
        _concat_body,
        grid_spec=grid_spec,
        out_shape=[
            jax.ShapeDtypeStruct((b, h, _K + s, dh), jnp.float32),
            jax.ShapeDtypeStruct((b, h, _K + s, dh), jnp.float32),
        ],
    )(topi_flat, k, v, mem_keys, mem_values)

    return (inputs, q, k_aug, v_aug, mask_aug, _K + s, positions_k)
